# baseline (speedup 1.0000x reference)
import functools

import jax
import jax.numpy as jnp
from jax.experimental import pallas as pl
from jax.experimental.pallas import tpu as pltpu

EPS = 1e-06
NEG_INF = float("-inf")


def _fused_kernel(x_ref, wr_ref, br_ref, we_ref, be_ref, wo_ref, bo_ref,
                  nw_ref, out_ref):
    xt = x_ref[...]
    E = wr_ref.shape[0]
    T = xt.shape[0]

    xb = xt.astype(jnp.bfloat16)
    logits = jax.lax.dot_general(
        xb, wr_ref[...].astype(jnp.bfloat16), (((1,), (1,)), ((), ())),
        preferred_element_type=jnp.float32) + br_ref[...]

    eidx = jax.lax.broadcasted_iota(jnp.int32, (T, E), 1)
    v0 = jnp.max(logits, axis=-1, keepdims=True)
    i0 = jnp.argmax(logits, axis=-1)
    masked = jnp.where(eidx == i0[:, None], NEG_INF, logits)
    v1 = jnp.max(masked, axis=-1, keepdims=True)
    i1 = jnp.argmax(masked, axis=-1)

    b = jnp.exp(v1 - v0)
    denom = 1.0 + b
    w0 = (1.0 / denom)[:, 0]
    w1 = (b / denom)[:, 0]

    comb = jnp.zeros(xt.shape, jnp.bfloat16)
    for e in range(E):
        h = jax.lax.dot_general(
            xb, we_ref[e], (((1,), (1,)), ((), ())),
            preferred_element_type=jnp.float32)
        h = (h + be_ref[e][None, :]).astype(jnp.bfloat16)
        h = h * jax.nn.sigmoid(h)
        wt = jnp.where(i0 == e, w0, 0.0) + jnp.where(i1 == e, w1, 0.0)
        comb = comb + wt.astype(jnp.bfloat16)[:, None] * h

    out = jax.lax.dot_general(
        comb, wo_ref[...], (((1,), (1,)), ((), ())),
        preferred_element_type=jnp.float32) + bo_ref[...]
    res = xt + out
    rms = jnp.sqrt(jnp.mean(res * res, axis=-1, keepdims=True) + EPS)
    out_ref[...] = nw_ref[...] * (res / rms)


@jax.jit
def kernel(x, Wr, br, We, be, Wo, bo, norm_w):
    B, S, D = x.shape
    E = Wr.shape[0]
    N = B * S
    T = 1024

    xf = x.reshape(N, D)
    we_bf = We.astype(jnp.bfloat16)
    wo_bf = Wo.astype(jnp.bfloat16)

    out = pl.pallas_call(
        _fused_kernel,
        grid=(N // T,),
        in_specs=[
            pl.BlockSpec((T, D), lambda i: (i, 0)),
            pl.BlockSpec((E, D), lambda i: (0, 0)),
            pl.BlockSpec((1, E), lambda i: (0, 0)),
            pl.BlockSpec((E, D, D), lambda i: (0, 0, 0)),
            pl.BlockSpec((E, D), lambda i: (0, 0)),
            pl.BlockSpec((D, D), lambda i: (0, 0)),
            pl.BlockSpec((1, D), lambda i: (0, 0)),
            pl.BlockSpec((1, D), lambda i: (0, 0)),
        ],
        out_specs=pl.BlockSpec((T, D), lambda i: (i, 0)),
        out_shape=jax.ShapeDtypeStruct((N, D), jnp.float32),
        compiler_params=pltpu.CompilerParams(
            dimension_semantics=("arbitrary",),
            vmem_limit_bytes=100 * 1024 * 1024,
            fuse_transposed_lhs_in_matmul=True,
        ),
    )(xf, Wr, br.reshape(1, E), we_bf, be, wo_bf, bo.reshape(1, D),
      norm_w.reshape(1, D))
    return out.reshape(B, S, D)
